# baseline (speedup 1.0000x reference)
import functools

import jax
import jax.numpy as jnp
from jax import lax
from jax.experimental import pallas as pl
from jax.experimental.pallas import tpu as pltpu
from jax.experimental.pallas import tpu_sc as plsc

D = 128
C = 128
NBUF = 5


@jax.jit
def _embed(idx3, table):
    NW, n_chunks, _ = idx3.shape

    mesh = plsc.VectorSubcoreMesh(core_axis_name="c", subcore_axis_name="s")
    info = plsc.get_sparse_core_info()
    NC = info.num_cores

    @functools.partial(
        pl.kernel,
        out_type=jax.ShapeDtypeStruct((NW * n_chunks, C, D), jnp.float32),
        mesh=mesh,
        scratch_types=(
            [pltpu.VMEM((n_chunks, C), jnp.int32)]
            + [pltpu.VMEM((C, D), jnp.float32) for _ in range(NBUF)]
            + [pltpu.SemaphoreType.DMA for _ in range(2 * NBUF)]
        ),
    )
    def body(idx_hbm, table_hbm, out_hbm, idx_v, *rest):
        bufs = rest[:NBUF]
        gsem = rest[NBUF:2 * NBUF]
        osem = rest[2 * NBUF:]
        wid = lax.axis_index("s") * NC + lax.axis_index("c")
        pltpu.sync_copy(idx_hbm.at[wid], idx_v)

        for b in range(NBUF):
            pltpu.async_copy(table_hbm.at[idx_v.at[b]], bufs[b], gsem[b])

        @pl.loop(0, n_chunks - NBUF, step=NBUF)
        def _steady(j0):
            for b in range(NBUF):
                j = j0 + b
                pltpu.make_async_copy(
                    table_hbm.at[idx_v.at[b]], bufs[b], gsem[b]).wait()
                pltpu.async_copy(bufs[b], out_hbm.at[wid * n_chunks + j], osem[b])
                pltpu.make_async_copy(
                    bufs[b], out_hbm.at[wid * n_chunks + j], osem[b]).wait()
                pltpu.async_copy(
                    table_hbm.at[idx_v.at[j + NBUF]], bufs[b], gsem[b])

        for b in range(NBUF):
            j = n_chunks - NBUF + b
            pltpu.make_async_copy(
                table_hbm.at[idx_v.at[b]], bufs[b], gsem[b]).wait()
            pltpu.async_copy(bufs[b], out_hbm.at[wid * n_chunks + j], osem[b])
            pltpu.make_async_copy(
                bufs[b], out_hbm.at[wid * n_chunks + j], osem[b]).wait()

    return body(idx3, table)


def kernel(input, table):
    B, H = input.shape
    N = B * H
    NW = 32
    n_per_w = N // NW
    n_chunks = n_per_w // C
    idx3 = input.reshape(NW, n_chunks, C).astype(jnp.int32)
    out = _embed(idx3, table)
    return out.reshape(B, H, D)


# speedup vs baseline: 1.7792x; 1.7792x over previous
import functools

import jax
import jax.numpy as jnp
from jax import lax
from jax.experimental import pallas as pl
from jax.experimental.pallas import tpu as pltpu
from jax.experimental.pallas import tpu_sc as plsc

D = 128
NB = 2
NBUF = 4


@jax.jit
def _embed(idx, table):
    B, H = idx.shape

    mesh = plsc.VectorSubcoreMesh(core_axis_name="c", subcore_axis_name="s")
    info = plsc.get_sparse_core_info()
    NC = info.num_cores
    NW = NC * info.num_subcores
    b_per_w = B // NW
    n_chunks = b_per_w // NB

    @functools.partial(
        pl.kernel,
        out_type=jax.ShapeDtypeStruct((B, H, D), jnp.float32),
        mesh=mesh,
        compiler_params=pltpu.CompilerParams(use_tc_tiling_on_sc=True),
        scratch_types=(
            [pltpu.VMEM((b_per_w, H), jnp.int32)]
            + [pltpu.VMEM((NB, H, D), jnp.float32) for _ in range(NBUF)]
            + [pltpu.SemaphoreType.DMA for _ in range(2 * NBUF)]
        ),
    )
    def body(idx_hbm, table_hbm, out_hbm, idx_v, *rest):
        bufs = rest[:NBUF]
        gsem = rest[NBUF:2 * NBUF]
        osem = rest[2 * NBUF:]
        wid = lax.axis_index("s") * NC + lax.axis_index("c")
        base = wid * b_per_w
        pltpu.sync_copy(idx_hbm.at[pl.ds(base, b_per_w)], idx_v)

        def start_gathers(c, r):
            for k in range(NB):
                pltpu.async_copy(
                    table_hbm.at[idx_v.at[c * NB + k]],
                    bufs[r].at[k], gsem[r])

        def wait_gathers(c, r):
            for k in range(NB):
                pltpu.make_async_copy(
                    table_hbm.at[idx_v.at[c * NB + k]],
                    bufs[r].at[k], gsem[r]).wait()

        def start_out(c, r):
            pltpu.async_copy(
                bufs[r], out_hbm.at[pl.ds(base + c * NB, NB)], osem[r])

        def wait_out(c, r):
            pltpu.make_async_copy(
                bufs[r], out_hbm.at[pl.ds(base + c * NB, NB)], osem[r]).wait()

        for r in range(NBUF):
            start_gathers(r, r)

        @pl.loop(0, n_chunks - NBUF, step=NBUF)
        def _steady(c0):
            for r in range(NBUF):
                c = c0 + r
                wait_gathers(c, r)
                start_out(c, r)
                wait_out(c, r)
                start_gathers(c + NBUF, r)

        for r in range(NBUF):
            c = n_chunks - NBUF + r
            wait_gathers(c, r)
            start_out(c, r)
            wait_out(c, r)

    return body(idx, table)


def kernel(input, table):
    return _embed(input.astype(jnp.int32), table)


# speedup vs baseline: 1.7927x; 1.0075x over previous
import functools

import jax
import jax.numpy as jnp
from jax import lax
from jax.experimental import pallas as pl
from jax.experimental.pallas import tpu as pltpu
from jax.experimental.pallas import tpu_sc as plsc

D = 128
NB = 2
NBUF = 8


@jax.jit
def _embed(idx, table):
    B, H = idx.shape

    mesh = plsc.VectorSubcoreMesh(core_axis_name="c", subcore_axis_name="s")
    info = plsc.get_sparse_core_info()
    NC = info.num_cores
    NW = NC * info.num_subcores
    b_per_w = B // NW
    n_chunks = b_per_w // NB

    @functools.partial(
        pl.kernel,
        out_type=jax.ShapeDtypeStruct((B, H, D), jnp.float32),
        mesh=mesh,
        compiler_params=pltpu.CompilerParams(use_tc_tiling_on_sc=True),
        scratch_types=(
            [pltpu.VMEM((b_per_w, H), jnp.int32)]
            + [pltpu.VMEM((NB, H, D), jnp.float32) for _ in range(NBUF)]
            + [pltpu.SemaphoreType.DMA for _ in range(2 * NBUF)]
        ),
    )
    def body(idx_hbm, table_hbm, out_hbm, idx_v, *rest):
        bufs = rest[:NBUF]
        gsem = rest[NBUF:2 * NBUF]
        osem = rest[2 * NBUF:]
        wid = lax.axis_index("s") * NC + lax.axis_index("c")
        base = wid * b_per_w
        pltpu.sync_copy(idx_hbm.at[pl.ds(base, b_per_w)], idx_v)

        def start_gathers(c, r):
            for k in range(NB):
                pltpu.async_copy(
                    table_hbm.at[idx_v.at[c * NB + k]],
                    bufs[r].at[k], gsem[r])

        def wait_gathers(c, r):
            for k in range(NB):
                pltpu.make_async_copy(
                    table_hbm.at[idx_v.at[c * NB + k]],
                    bufs[r].at[k], gsem[r]).wait()

        def start_out(c, r):
            pltpu.async_copy(
                bufs[r], out_hbm.at[pl.ds(base + c * NB, NB)], osem[r])

        def wait_out(c, r):
            pltpu.make_async_copy(
                bufs[r], out_hbm.at[pl.ds(base + c * NB, NB)], osem[r]).wait()

        for r in range(NBUF):
            start_gathers(r, r)

        @pl.loop(0, n_chunks - NBUF, step=NBUF)
        def _steady(c0):
            for r in range(NBUF):
                c = c0 + r
                wait_gathers(c, r)
                start_out(c, r)
                wait_out(c, r)
                start_gathers(c + NBUF, r)

        for r in range(NBUF):
            c = n_chunks - NBUF + r
            wait_gathers(c, r)
            start_out(c, r)
            wait_out(c, r)

    return body(idx, table)


def kernel(input, table):
    return _embed(input.astype(jnp.int32), table)
